# baseline (speedup 1.0000x reference)
import functools

import jax
import jax.numpy as jnp
from jax import lax
from jax.experimental import pallas as pl
from jax.experimental.pallas import tpu as pltpu
from jax.experimental.pallas import tpu_sc as plsc

S = 1024
N = 50
NMAX = 64
D = 128
VOCAB = 100000
T = 3 * N
TP = 152
NC = 2
NS = 16
NW = NC * NS
BPW = S // NW
G0, G1 = 80, 72

_mesh = plsc.VectorSubcoreMesh(core_axis_name="c", subcore_axis_name="s")


@functools.partial(
    pl.kernel,
    out_type=jax.ShapeDtypeStruct((S, T, 2 * NMAX + D), jnp.float32),
    mesh=_mesh,
    scratch_types=[
        pltpu.VMEM((2 * BPW, TP // 2), jnp.int32),
        [pltpu.VMEM((TP, D), jnp.float32) for _ in range(3)],
        pltpu.VMEM((T, D), jnp.float32),
        [pltpu.SemaphoreType.DMA for _ in range(7)],
    ],
)
def _embed_sc(idx, zeros_h, tab, out, ibuf, dbufs, zbuf, sems):
    wid = lax.axis_index("s") * NC + lax.axis_index("c")
    gsems = sems[0:3]
    dsems = sems[3:6]
    zsem = sems[6]
    b0 = wid * BPW
    pltpu.sync_copy(idx.at[pl.ds(2 * b0, 2 * BPW), :], ibuf)
    pltpu.sync_copy(zeros_h, zbuf)

    zds = [pltpu.async_copy(zbuf, out.at[b0 + j, :, pl.ds(0, D)], zsem)
           for j in range(0)]

    H = TP // 2

    def fire_gathers(j):
        p = j % 3
        return (
            pltpu.async_copy(tab.at[ibuf.at[2 * j]],
                             dbufs[p].at[pl.ds(0, H)], gsems[p]),
            pltpu.async_copy(tab.at[ibuf.at[2 * j + 1]],
                             dbufs[p].at[pl.ds(H, H)], gsems[p]),
        )

    def fire_scatters(j):
        p = j % 3
        b = b0 + j
        return tuple(
            pltpu.async_copy(dbufs[p].at[pl.ds(r0, nr)],
                             out.at[b, pl.ds(r0, nr), pl.ds(D, D)], dsems[p])
            for r0, nr in ((0, 48), (48, 48), (96, 54)))

    gds = {0: fire_gathers(0)}
    sds = {}
    for j in range(BPW):
        if j + 1 < BPW:
            gds[j + 1] = fire_gathers(j + 1)
    for j in range(BPW):
        for d in gds[j]:
            d.wait()
    del sds
    for d in zds:
        d.wait()


def kernel(examples, labels, embeddings, label_embeddings):
    tab = jnp.concatenate([embeddings, label_embeddings], axis=0)
    trip = jnp.stack(
        [examples[:, 0::2], examples[:, 1::2], labels[:, :-1] + VOCAB],
        axis=2)
    idx = jnp.pad(trip.reshape(S, T), ((0, 0), (0, TP - T)))
    idx = idx.reshape(2 * S, TP // 2)
    zeros_h = jnp.zeros((T, D), jnp.float32)
    return _embed_sc(idx, zeros_h, tab)


# speedup vs baseline: 2.3630x; 2.3630x over previous
import functools

import jax
import jax.numpy as jnp
from jax import lax
from jax.experimental import pallas as pl
from jax.experimental.pallas import tpu as pltpu
from jax.experimental.pallas import tpu_sc as plsc

S = 1024
N = 50
NMAX = 64
D = 128
T = 3 * N
NC = 2
NS = 16
NW = NC * NS
BG = 64
NG = S // BG
NIT = T * NG // NW
PAR = 6
LAG = 2

_mesh = plsc.VectorSubcoreMesh(core_axis_name="c", subcore_axis_name="s")


@functools.partial(
    pl.kernel,
    out_type=jax.ShapeDtypeStruct((T, S, 2 * NMAX + D), jnp.float32),
    mesh=_mesh,
    scratch_types=[
        pltpu.VMEM((NIT * BG,), jnp.int32),
        [pltpu.VMEM((BG, D), jnp.float32) for _ in range(PAR)],
        pltpu.VMEM((BG, D), jnp.float32),
        [pltpu.SemaphoreType.DMA for _ in range(2 * PAR + 1)],
    ],
)
def _embed_sc(idx, zeros_h, emb, lemb, out, ibuf, dbufs, zbuf, sems):
    wid = lax.axis_index("s") * NC + lax.axis_index("c")
    gsems = sems[0:PAR]
    dsems = sems[PAR:2 * PAR]
    zsem = sems[2 * PAR]
    g = wid // 2
    tbase = (wid % 2) * NIT
    bcol = g * BG
    pltpu.sync_copy(idx.at[pl.ds(wid * NIT * BG, NIT * BG)], ibuf)
    pltpu.sync_copy(zeros_h, zbuf)

    def fire_gather(li):
        src = lemb if li % 3 == 2 else emb
        p = li % PAR
        return pltpu.async_copy(
            src.at[ibuf.at[pl.ds(li * BG, BG)]], dbufs[p], gsems[p])

    def fire_scatters(li):
        t = tbase + li
        p = li % PAR
        return (
            pltpu.async_copy(dbufs[p],
                             out.at[t, pl.ds(bcol, BG), pl.ds(D, D)],
                             dsems[p]),
            pltpu.async_copy(zbuf,
                             out.at[t, pl.ds(bcol, BG), pl.ds(0, D)],
                             zsem),
        )

    gds, sds, zds = {}, {}, []
    for i in range(NIT + LAG):
        if i < NIT:
            if i >= PAR:
                sds[i - PAR].wait()
            gds[i] = fire_gather(i)
        k = i - LAG
        if k >= 0:
            gds[k].wait()
            sd, zd = fire_scatters(k)
            sds[k] = sd
            zds.append(zd)
    for k in range(NIT - PAR, NIT):
        sds[k].wait()
    for d in zds:
        d.wait()


def kernel(examples, labels, embeddings, label_embeddings):
    trip = jnp.stack(
        [examples[:, 0::2], examples[:, 1::2], labels[:, :-1]], axis=2)
    idx = (trip.reshape(S, T)
           .reshape(NG, BG, T)
           .transpose(0, 2, 1)
           .reshape(-1))
    zeros_h = jnp.zeros((BG, D), jnp.float32)
    out = _embed_sc(idx, zeros_h, embeddings, label_embeddings)
    return jnp.transpose(out, (1, 0, 2))
